# initial kernel scaffold (unmeasured)
import jax
import jax.numpy as jnp
from jax import lax
from jax.experimental import pallas as pl
from jax.experimental.pallas import tpu as pltpu

P = 4


def kernel(x, w_mat):
    m, kp = x.shape
    _, n = w_mat.shape
    mc = m // P
    hn = n // 2
    sub = mc // 2

    def body(x_hbm, w_ref, out_ref, commA, commB, xc, xsem, osem,
             sendA, recvA, sendB, recvB):
        p = lax.axis_index("i")
        right = lax.rem(p + 1, P)
        left = lax.rem(p + P - 1, P)

        bar = pltpu.get_barrier_semaphore()

        def hop_barrier():
            for nbr in (left, right):
                pl.semaphore_signal(
                    bar, inc=1, device_id=(nbr,),
                    device_id_type=pl.DeviceIdType.MESH)
            pl.semaphore_wait(bar, 2)

        def partial(c, half, comm, slot, accumulate):
            for s in range(mc // sub):
                cp = pltpu.make_async_copy(
                    x_hbm.at[pl.ds(c * mc + s * sub, sub), :], xc, xsem)
                cp.start()
                cp.wait()
                d = jnp.dot(xc[...], w_ref[:, half * hn:(half + 1) * hn],
                            preferred_element_type=jnp.bfloat16)
                rows = pl.ds(s * sub, sub)
                if accumulate:
                    comm[slot, rows, :] = comm[slot, rows, :] + d
                else:
                    comm[slot, rows, :] = d

        def store(comm, slot, c, half):
            cp = pltpu.make_async_copy(
                comm.at[slot],
                out_ref.at[pl.ds(c * mc, mc), pl.ds(half * hn, hn)],
                osem)
            cp.start()
            cp.wait()

        def hop(h):
            s_, r_ = h % 2, (h + 1) % 2
            hop_barrier()
            ra = pltpu.make_async_remote_copy(
                src_ref=commA.at[s_], dst_ref=commA.at[r_],
                send_sem=sendA.at[s_], recv_sem=recvA.at[r_],
                device_id=(right,), device_id_type=pl.DeviceIdType.MESH)
            rb = pltpu.make_async_remote_copy(
                src_ref=commB.at[s_], dst_ref=commB.at[r_],
                send_sem=sendB.at[s_], recv_sem=recvB.at[r_],
                device_id=(left,), device_id_type=pl.DeviceIdType.MESH)
            ra.start()
            rb.start()
            ra.wait()
            rb.wait()
            return r_

        partial(p, 0, commA, 0, False)
        partial(p, 1, commB, 0, False)

        for h in range(P - 1):
            r_ = hop(h)
            cA = lax.rem(p + P - h - 1, P)
            cB = lax.rem(p + h + 1, P)
            partial(cA, 0, commA, r_, True)
            partial(cB, 1, commB, r_, True)

        store(commA, 1, right, 0)
        store(commB, 1, left, 1)

        for k in range(P - 1):
            r_ = hop(k + P - 1)
            cA = lax.rem(p + P - k, P)
            cB = lax.rem(p + k, P)
            store(commA, r_, cA, 0)
            store(commB, r_, cB, 1)

    return pl.pallas_call(
        body,
        out_shape=jax.ShapeDtypeStruct((m, n), jnp.bfloat16),
        in_specs=[pl.BlockSpec(memory_space=pltpu.ANY),
                  pl.BlockSpec(memory_space=pltpu.VMEM)],
        out_specs=pl.BlockSpec(memory_space=pltpu.ANY),
        scratch_shapes=[
            pltpu.VMEM((2, mc, hn), jnp.bfloat16),
            pltpu.VMEM((2, mc, hn), jnp.bfloat16),
            pltpu.VMEM((sub, kp), jnp.bfloat16),
            pltpu.SemaphoreType.DMA,
            pltpu.SemaphoreType.DMA,
            pltpu.SemaphoreType.DMA((2,)),
            pltpu.SemaphoreType.DMA((2,)),
            pltpu.SemaphoreType.DMA((2,)),
            pltpu.SemaphoreType.DMA((2,)),
        ],
        compiler_params=pltpu.CompilerParams(collective_id=0),
    )(x, w_mat)


# baseline (device time: 861551 ns/iter reference)
import jax
import jax.numpy as jnp
from jax import lax
from jax.experimental import pallas as pl
from jax.experimental.pallas import tpu as pltpu

P = 4
G = 2


def kernel(x, w_mat):
    x = x.astype(jnp.bfloat16)
    w_mat = w_mat.astype(jnp.bfloat16)
    m, kp = x.shape
    _, n = w_mat.shape
    gm = m // G
    mc = gm // P
    hn = n // 2
    nt = hn // 2

    def body(x_hbm, w_ref, out_ref, commA, commB, xc, xsem, osem,
             sendA, recvA, sendB, recvB):
        p = lax.axis_index("i")
        right = lax.rem(p + 1, P)
        left = lax.rem(p + P - 1, P)

        bar = pltpu.get_barrier_semaphore()

        def hop_barrier():
            for nbr in (left, right):
                pl.semaphore_signal(
                    bar, inc=1, device_id=(nbr,),
                    device_id_type=pl.DeviceIdType.MESH)
            pl.semaphore_wait(bar, 2)

        def partial(g, c, half, comm, slot, accumulate):
            cp = pltpu.make_async_copy(
                x_hbm.at[pl.ds(g * gm + c * mc, mc), :], xc, xsem)
            cp.start()
            cp.wait()
            for t in range(hn // nt):
                d = jnp.dot(xc[...],
                            w_ref[:, half * hn + t * nt:
                                  half * hn + (t + 1) * nt],
                            preferred_element_type=jnp.float32
                            ).astype(jnp.bfloat16)
                cols = pl.ds(t * nt, nt)
                if accumulate:
                    comm[slot, :, cols] = comm[slot, :, cols] + d
                else:
                    comm[slot, :, cols] = d

        def store(g, comm, slot, c, half):
            cp = pltpu.make_async_copy(
                comm.at[slot],
                out_ref.at[pl.ds(g * gm + c * mc, mc),
                           pl.ds(half * hn, hn)],
                osem)
            cp.start()
            cp.wait()

        def hop(h):
            s_, r_ = h % 2, (h + 1) % 2
            hop_barrier()
            ra = pltpu.make_async_remote_copy(
                src_ref=commA.at[s_], dst_ref=commA.at[r_],
                send_sem=sendA.at[s_], recv_sem=recvA.at[r_],
                device_id=(right,), device_id_type=pl.DeviceIdType.MESH)
            rb = pltpu.make_async_remote_copy(
                src_ref=commB.at[s_], dst_ref=commB.at[r_],
                send_sem=sendB.at[s_], recv_sem=recvB.at[r_],
                device_id=(left,), device_id_type=pl.DeviceIdType.MESH)
            ra.start()
            rb.start()
            ra.wait()
            rb.wait()
            return r_

        for g in range(G):
            partial(g, p, 0, commA, 0, False)
            partial(g, p, 1, commB, 0, False)

            for h in range(P - 1):
                r_ = hop(h)
                cA = lax.rem(p + P - h - 1, P)
                cB = lax.rem(p + h + 1, P)
                partial(g, cA, 0, commA, r_, True)
                partial(g, cB, 1, commB, r_, True)

            store(g, commA, 1, right, 0)
            store(g, commB, 1, left, 1)

            for k in range(P - 1):
                r_ = hop(k + P - 1)
                cA = lax.rem(p + P - k, P)
                cB = lax.rem(p + k, P)
                store(g, commA, r_, cA, 0)
                store(g, commB, r_, cB, 1)

    return pl.pallas_call(
        body,
        out_shape=jax.ShapeDtypeStruct((m, n), jnp.bfloat16),
        in_specs=[pl.BlockSpec(memory_space=pl.ANY),
                  pl.BlockSpec(memory_space=pltpu.MemorySpace.VMEM)],
        out_specs=pl.BlockSpec(memory_space=pl.ANY),
        scratch_shapes=[
            pltpu.VMEM((2, mc, hn), jnp.bfloat16),
            pltpu.VMEM((2, mc, hn), jnp.bfloat16),
            pltpu.VMEM((mc, kp), jnp.bfloat16),
            pltpu.SemaphoreType.DMA,
            pltpu.SemaphoreType.DMA,
            pltpu.SemaphoreType.DMA((2,)),
            pltpu.SemaphoreType.DMA((2,)),
            pltpu.SemaphoreType.DMA((2,)),
            pltpu.SemaphoreType.DMA((2,)),
        ],
        compiler_params=pltpu.CompilerParams(collective_id=0),
    )(x, w_mat)


# device time: 718276 ns/iter; 1.1995x vs baseline; 1.1995x over previous
import jax
import jax.numpy as jnp
from jax import lax
from jax.experimental import pallas as pl
from jax.experimental.pallas import tpu as pltpu

P = 4
G = 2
NT = 512


def kernel(x, w_mat):
    x = x.astype(jnp.bfloat16)
    w_mat = w_mat.astype(jnp.bfloat16)
    m, kp = x.shape
    _, n = w_mat.shape
    gm = m // G
    mc = gm // P
    hn = n // 2

    def body(x_hbm, w_ref, out_ref, commA, commB, pA, pB, xc,
             xsems, osems, sendA, recvA, sendB, recvB):
        p = lax.axis_index("i")
        right = lax.rem(p + 1, P)
        left = lax.rem(p + P - 1, P)

        bar = pltpu.get_barrier_semaphore()
        pending = []

        def hop_barrier():
            for nbr in (left, right):
                pl.semaphore_signal(
                    bar, inc=1, device_id=(nbr,),
                    device_id_type=pl.DeviceIdType.MESH)
            pl.semaphore_wait(bar, 2)

        def flush():
            while pending:
                pending.pop().wait()

        def fetch_x(g, c, xslot):
            cp = pltpu.make_async_copy(
                x_hbm.at[pl.ds(g * gm + c * mc, mc), :],
                xc.at[xslot], xsems.at[xslot])
            cp.start()
            return cp

        def dots(half, xslot, dst):
            def tbody(t, carry):
                d = jnp.dot(xc[xslot],
                            w_ref[:, pl.ds(half * hn + t * NT, NT)],
                            preferred_element_type=jnp.float32
                            ).astype(jnp.bfloat16)
                dst[:, pl.ds(t * NT, NT)] = d
                return carry
            lax.fori_loop(0, hn // NT, tbody, 0)

        def add_in(comm, slot, pbuf):
            def tbody(t, carry):
                cols = pl.ds(t * NT, NT)
                comm[slot, :, cols] = comm[slot, :, cols] + pbuf[:, cols]
                return carry
            lax.fori_loop(0, hn // NT, tbody, 0)

        def store(comm, slot, g, c, half, osem):
            cp = pltpu.make_async_copy(
                comm.at[slot],
                out_ref.at[pl.ds(g * gm + c * mc, mc),
                           pl.ds(half * hn, hn)],
                osem)
            cp.start()
            pending.append(cp)

        def make_hop(h):
            s_, r_ = h % 2, (h + 1) % 2
            ra = pltpu.make_async_remote_copy(
                src_ref=commA.at[s_], dst_ref=commA.at[r_],
                send_sem=sendA.at[s_], recv_sem=recvA.at[r_],
                device_id=(right,), device_id_type=pl.DeviceIdType.MESH)
            rb = pltpu.make_async_remote_copy(
                src_ref=commB.at[s_], dst_ref=commB.at[r_],
                send_sem=sendB.at[s_], recv_sem=recvB.at[r_],
                device_id=(left,), device_id_type=pl.DeviceIdType.MESH)
            return ra, rb, r_

        for g in range(G):
            cpx = fetch_x(g, p, 0)
            cpx.wait()
            dots(0, 0, commA.at[0])
            dots(1, 0, commB.at[0])

            for h in range(P - 1):
                hop_barrier()
                ra, rb, r_ = make_hop(h)
                ra.start()
                rb.start()
                cA = lax.rem(p + P - h - 1, P)
                cB = lax.rem(p + h + 1, P)
                ca_x = fetch_x(g, cA, 0)
                cb_x = fetch_x(g, cB, 1)
                ca_x.wait()
                dots(0, 0, pA)
                cb_x.wait()
                dots(1, 1, pB)
                ra.wait_recv()
                add_in(commA, r_, pA)
                rb.wait_recv()
                add_in(commB, r_, pB)
                ra.wait_send()
                rb.wait_send()

            flush()
            store(commA, 1, g, right, 0, osems.at[0])
            store(commB, 1, g, left, 1, osems.at[1])

            for k in range(P - 1):
                hop_barrier()
                ra, rb, r_ = make_hop(k + P - 1)
                ra.start()
                rb.start()
                ra.wait_recv()
                rb.wait_recv()
                cA = lax.rem(p + P - k, P)
                cB = lax.rem(p + k, P)
                flush()
                store(commA, r_, g, cA, 0, osems.at[0])
                store(commB, r_, g, cB, 1, osems.at[1])
                ra.wait_send()
                rb.wait_send()

            flush()

    return pl.pallas_call(
        body,
        out_shape=jax.ShapeDtypeStruct((m, n), jnp.bfloat16),
        in_specs=[pl.BlockSpec(memory_space=pl.ANY),
                  pl.BlockSpec(memory_space=pltpu.MemorySpace.VMEM)],
        out_specs=pl.BlockSpec(memory_space=pl.ANY),
        scratch_shapes=[
            pltpu.VMEM((2, mc, hn), jnp.bfloat16),
            pltpu.VMEM((2, mc, hn), jnp.bfloat16),
            pltpu.VMEM((mc, hn), jnp.bfloat16),
            pltpu.VMEM((mc, hn), jnp.bfloat16),
            pltpu.VMEM((2, mc, kp), jnp.bfloat16),
            pltpu.SemaphoreType.DMA((2,)),
            pltpu.SemaphoreType.DMA((2,)),
            pltpu.SemaphoreType.DMA((2,)),
            pltpu.SemaphoreType.DMA((2,)),
            pltpu.SemaphoreType.DMA((2,)),
            pltpu.SemaphoreType.DMA((2,)),
        ],
        compiler_params=pltpu.CompilerParams(collective_id=0),
    )(x, w_mat)


# device time: 690515 ns/iter; 1.2477x vs baseline; 1.0402x over previous
import jax
import jax.numpy as jnp
from jax import lax
from jax.experimental import pallas as pl
from jax.experimental.pallas import tpu as pltpu

P = 4
G = 2
NT = 512
S = 2


def kernel(x, w_mat):
    x = x.astype(jnp.bfloat16)
    w_mat = w_mat.astype(jnp.bfloat16)
    m, kp = x.shape
    _, n = w_mat.shape
    gm = m // G
    mc = gm // P
    hn = n // 2
    sr = mc // S

    def body(x_hbm, w_ref, out_ref, commA, commB, pA, pB, xc,
             xsems, osems, csems, sendA, recvA, sendB, recvB):
        p = lax.axis_index("i")
        right = lax.rem(p + 1, P)
        left = lax.rem(p + P - 1, P)

        bar = pltpu.get_barrier_semaphore()
        pending = []

        def hop_barrier():
            for nbr in (left, right):
                pl.semaphore_signal(
                    bar, inc=1, device_id=(nbr,),
                    device_id_type=pl.DeviceIdType.MESH)
            pl.semaphore_wait(bar, 2)

        def flush():
            while pending:
                pending.pop().wait()

        def fetch_x(g, c, xslot):
            cp = pltpu.make_async_copy(
                x_hbm.at[pl.ds(g * gm + c * mc, mc), :],
                xc.at[xslot], xsems.at[xslot])
            cp.start()
            return cp

        def dots(half, xslot, dst):
            def tbody(t, carry):
                d = jnp.dot(xc[xslot],
                            w_ref[:, pl.ds(half * hn + t * NT, NT)],
                            preferred_element_type=jnp.float32
                            ).astype(jnp.bfloat16)
                dst[:, pl.ds(t * NT, NT)] = d
                return carry
            lax.fori_loop(0, hn // NT, tbody, 0)

        def add_sub(comm, slot, pbuf, sub):
            rows = pl.ds(sub * sr, sr)
            def tbody(t, carry):
                cols = pl.ds(t * 1024, 1024)
                comm[slot, rows, cols] = (
                    comm[slot, rows, cols] + pbuf[rows, cols])
                return carry
            lax.fori_loop(0, hn // 1024, tbody, 0)

        def store(comm, slot, g, c, half, osem):
            cp = pltpu.make_async_copy(
                comm.at[slot],
                out_ref.at[pl.ds(g * gm + c * mc, mc),
                           pl.ds(half * hn, hn)],
                osem)
            cp.start()
            pending.append(cp)

        def rs_rdmas(h):
            s_, r_ = h % 2, (h + 1) % 2
            ras, rbs = [], []
            for u in range(S):
                rows = pl.ds(u * sr, sr)
                ras.append(pltpu.make_async_remote_copy(
                    src_ref=commA.at[s_, rows], dst_ref=commA.at[r_, rows],
                    send_sem=sendA.at[s_, u], recv_sem=recvA.at[r_, u],
                    device_id=(right,),
                    device_id_type=pl.DeviceIdType.MESH))
                rbs.append(pltpu.make_async_remote_copy(
                    src_ref=commB.at[s_, rows], dst_ref=commB.at[r_, rows],
                    send_sem=sendB.at[s_, u], recv_sem=recvB.at[r_, u],
                    device_id=(left,),
                    device_id_type=pl.DeviceIdType.MESH))
            return ras, rbs, r_

        def ag_rdmas(h):
            s_, r_ = h % 2, (h + 1) % 2
            ra = pltpu.make_async_remote_copy(
                src_ref=commA.at[s_], dst_ref=commA.at[r_],
                send_sem=sendA.at[s_, 0], recv_sem=recvA.at[r_, 0],
                device_id=(right,), device_id_type=pl.DeviceIdType.MESH)
            rb = pltpu.make_async_remote_copy(
                src_ref=commB.at[s_], dst_ref=commB.at[r_],
                send_sem=sendB.at[s_, 0], recv_sem=recvB.at[r_, 0],
                device_id=(left,), device_id_type=pl.DeviceIdType.MESH)
            return ra, rb, r_

        def pre_phase(g):
            cpx = fetch_x(g, p, 0)
            cpx.wait()
            dots(0, 0, pA)
            dots(1, 0, pB)

        def load_send_slots():
            ca = pltpu.make_async_copy(pA, commA.at[0], csems.at[0])
            cb = pltpu.make_async_copy(pB, commB.at[0], csems.at[1])
            ca.start()
            cb.start()
            ca.wait()
            cb.wait()

        pre_phase(0)

        for g in range(G):
            load_send_slots()

            for h in range(P - 1):
                hop_barrier()
                ras, rbs, r_ = rs_rdmas(h)
                for r in ras + rbs:
                    r.start()
                cA = lax.rem(p + P - h - 1, P)
                cB = lax.rem(p + h + 1, P)
                ca_x = fetch_x(g, cA, 0)
                cb_x = fetch_x(g, cB, 1)
                ca_x.wait()
                dots(0, 0, pA)
                cb_x.wait()
                dots(1, 1, pB)
                for u in range(S):
                    ras[u].wait_recv()
                    add_sub(commA, r_, pA, u)
                    rbs[u].wait_recv()
                    add_sub(commB, r_, pB, u)
                for r in ras + rbs:
                    r.wait_send()

            store(commA, 1, g, right, 0, osems.at[0])
            store(commB, 1, g, left, 1, osems.at[1])

            for k in range(P - 1):
                hop_barrier()
                ra, rb, r_ = ag_rdmas(k + P - 1)
                ra.start()
                rb.start()
                if k == P - 2 and g < G - 1:
                    pre_phase(g + 1)
                ra.wait_recv()
                rb.wait_recv()
                cA = lax.rem(p + P - k, P)
                cB = lax.rem(p + k, P)
                flush()
                store(commA, r_, g, cA, 0, osems.at[0])
                store(commB, r_, g, cB, 1, osems.at[1])
                ra.wait_send()
                rb.wait_send()

            flush()

    return pl.pallas_call(
        body,
        out_shape=jax.ShapeDtypeStruct((m, n), jnp.bfloat16),
        in_specs=[pl.BlockSpec(memory_space=pl.ANY),
                  pl.BlockSpec(memory_space=pltpu.MemorySpace.VMEM)],
        out_specs=pl.BlockSpec(memory_space=pl.ANY),
        scratch_shapes=[
            pltpu.VMEM((2, mc, hn), jnp.bfloat16),
            pltpu.VMEM((2, mc, hn), jnp.bfloat16),
            pltpu.VMEM((mc, hn), jnp.bfloat16),
            pltpu.VMEM((mc, hn), jnp.bfloat16),
            pltpu.VMEM((2, mc, kp), jnp.bfloat16),
            pltpu.SemaphoreType.DMA((2,)),
            pltpu.SemaphoreType.DMA((2,)),
            pltpu.SemaphoreType.DMA((2,)),
            pltpu.SemaphoreType.DMA((2, S)),
            pltpu.SemaphoreType.DMA((2, S)),
            pltpu.SemaphoreType.DMA((2, S)),
            pltpu.SemaphoreType.DMA((2, S)),
        ],
        compiler_params=pltpu.CompilerParams(collective_id=0),
    )(x, w_mat)


# device time: 689526 ns/iter; 1.2495x vs baseline; 1.0014x over previous
import jax
import jax.numpy as jnp
from jax import lax
from jax.experimental import pallas as pl
from jax.experimental.pallas import tpu as pltpu

P = 4
G = 2
NT = 512
S = 4


def kernel(x, w_mat):
    x = x.astype(jnp.bfloat16)
    w_mat = w_mat.astype(jnp.bfloat16)
    m, kp = x.shape
    _, n = w_mat.shape
    gm = m // G
    mc = gm // P
    hn = n // 2
    sr = mc // S

    def body(x_hbm, w_ref, out_ref, commA, commB, pA, pB, xc,
             xsems, osems, csems, sendA, recvA, sendB, recvB):
        p = lax.axis_index("i")
        right = lax.rem(p + 1, P)
        left = lax.rem(p + P - 1, P)

        bar = pltpu.get_barrier_semaphore()
        pending = []

        def hop_barrier():
            for nbr in (left, right):
                pl.semaphore_signal(
                    bar, inc=1, device_id=(nbr,),
                    device_id_type=pl.DeviceIdType.MESH)
            pl.semaphore_wait(bar, 2)

        def flush():
            while pending:
                pending.pop().wait()

        def fetch_x(g, c, xslot):
            cp = pltpu.make_async_copy(
                x_hbm.at[pl.ds(g * gm + c * mc, mc), :],
                xc.at[xslot], xsems.at[xslot])
            cp.start()
            return cp

        def dots(half, xslot, dst):
            def tbody(t, carry):
                d = jnp.dot(xc[xslot],
                            w_ref[:, pl.ds(half * hn + t * NT, NT)],
                            preferred_element_type=jnp.float32
                            ).astype(jnp.bfloat16)
                dst[:, pl.ds(t * NT, NT)] = d
                return carry
            lax.fori_loop(0, hn // NT, tbody, 0)

        def add_sub(comm, slot, pbuf, sub):
            rows = pl.ds(sub * sr, sr)
            def tbody(t, carry):
                cols = pl.ds(t * 1024, 1024)
                comm[slot, rows, cols] = (
                    comm[slot, rows, cols] + pbuf[rows, cols])
                return carry
            lax.fori_loop(0, hn // 1024, tbody, 0)

        def store(comm, slot, g, c, half, osem):
            cp = pltpu.make_async_copy(
                comm.at[slot],
                out_ref.at[pl.ds(g * gm + c * mc, mc),
                           pl.ds(half * hn, hn)],
                osem)
            cp.start()
            pending.append(cp)

        def rs_rdmas(h):
            s_, r_ = h % 2, (h + 1) % 2
            ras, rbs = [], []
            for u in range(S):
                rows = pl.ds(u * sr, sr)
                ras.append(pltpu.make_async_remote_copy(
                    src_ref=commA.at[s_, rows], dst_ref=commA.at[r_, rows],
                    send_sem=sendA.at[s_, u], recv_sem=recvA.at[r_, u],
                    device_id=(right,),
                    device_id_type=pl.DeviceIdType.MESH))
                rbs.append(pltpu.make_async_remote_copy(
                    src_ref=commB.at[s_, rows], dst_ref=commB.at[r_, rows],
                    send_sem=sendB.at[s_, u], recv_sem=recvB.at[r_, u],
                    device_id=(left,),
                    device_id_type=pl.DeviceIdType.MESH))
            return ras, rbs, r_

        def ag_rdmas(h):
            s_, r_ = h % 2, (h + 1) % 2
            ra = pltpu.make_async_remote_copy(
                src_ref=commA.at[s_], dst_ref=commA.at[r_],
                send_sem=sendA.at[s_, 0], recv_sem=recvA.at[r_, 0],
                device_id=(right,), device_id_type=pl.DeviceIdType.MESH)
            rb = pltpu.make_async_remote_copy(
                src_ref=commB.at[s_], dst_ref=commB.at[r_],
                send_sem=sendB.at[s_, 0], recv_sem=recvB.at[r_, 0],
                device_id=(left,), device_id_type=pl.DeviceIdType.MESH)
            return ra, rb, r_

        def pre_phase(g):
            cpx = fetch_x(g, p, 0)
            cpx.wait()
            dots(0, 0, pA)
            dots(1, 0, pB)

        def load_send_slots():
            ca = pltpu.make_async_copy(pA, commA.at[0], csems.at[0])
            cb = pltpu.make_async_copy(pB, commB.at[0], csems.at[1])
            ca.start()
            cb.start()
            ca.wait()
            cb.wait()

        pre_phase(0)

        for g in range(G):
            load_send_slots()

            for h in range(P - 1):
                hop_barrier()
                ras, rbs, r_ = rs_rdmas(h)
                for r in ras + rbs:
                    r.start()
                cA = lax.rem(p + P - h - 1, P)
                cB = lax.rem(p + h + 1, P)
                ca_x = fetch_x(g, cA, 0)
                cb_x = fetch_x(g, cB, 1)
                ca_x.wait()
                dots(0, 0, pA)
                cb_x.wait()
                dots(1, 1, pB)
                for u in range(S):
                    ras[u].wait_recv()
                    add_sub(commA, r_, pA, u)
                    rbs[u].wait_recv()
                    add_sub(commB, r_, pB, u)
                for r in ras + rbs:
                    r.wait_send()

            store(commA, 1, g, right, 0, osems.at[0])
            store(commB, 1, g, left, 1, osems.at[1])

            for k in range(P - 1):
                hop_barrier()
                ra, rb, r_ = ag_rdmas(k + P - 1)
                ra.start()
                rb.start()
                if k == P - 2 and g < G - 1:
                    pre_phase(g + 1)
                ra.wait_recv()
                rb.wait_recv()
                cA = lax.rem(p + P - k, P)
                cB = lax.rem(p + k, P)
                flush()
                store(commA, r_, g, cA, 0, osems.at[0])
                store(commB, r_, g, cB, 1, osems.at[1])
                ra.wait_send()
                rb.wait_send()

            flush()

    return pl.pallas_call(
        body,
        out_shape=jax.ShapeDtypeStruct((m, n), jnp.bfloat16),
        in_specs=[pl.BlockSpec(memory_space=pl.ANY),
                  pl.BlockSpec(memory_space=pltpu.MemorySpace.VMEM)],
        out_specs=pl.BlockSpec(memory_space=pl.ANY),
        scratch_shapes=[
            pltpu.VMEM((2, mc, hn), jnp.bfloat16),
            pltpu.VMEM((2, mc, hn), jnp.bfloat16),
            pltpu.VMEM((mc, hn), jnp.bfloat16),
            pltpu.VMEM((mc, hn), jnp.bfloat16),
            pltpu.VMEM((2, mc, kp), jnp.bfloat16),
            pltpu.SemaphoreType.DMA((2,)),
            pltpu.SemaphoreType.DMA((2,)),
            pltpu.SemaphoreType.DMA((2,)),
            pltpu.SemaphoreType.DMA((2, S)),
            pltpu.SemaphoreType.DMA((2, S)),
            pltpu.SemaphoreType.DMA((2, S)),
            pltpu.SemaphoreType.DMA((2, S)),
        ],
        compiler_params=pltpu.CompilerParams(collective_id=0),
    )(x, w_mat)
